# baseline (speedup 1.0000x reference)
import jax
import jax.numpy as jnp
from jax.experimental import pallas as pl

_E = 16
_BLK = 2048


def _moe_body(x_ref, w_ref, b_ref, o_ref):
    xb = x_ref[...]
    w = w_ref[...]
    acc = jnp.dot(xb, w, preferred_element_type=jnp.float32)
    acc = acc + b_ref[...]
    nz = acc[:, :_E]
    eo = acc[:, _E:]
    iota = jax.lax.broadcasted_iota(jnp.int32, nz.shape, 1)
    v1 = jnp.max(nz, axis=1, keepdims=True)
    i1 = jnp.min(jnp.where(nz == v1, iota, _E), axis=1, keepdims=True)
    m1 = iota == i1
    masked = jnp.where(m1, -jnp.inf, nz)
    v2 = jnp.max(masked, axis=1, keepdims=True)
    i2 = jnp.min(jnp.where(masked == v2, iota, _E), axis=1, keepdims=True)
    m2 = iota == i2
    t = jnp.exp(v2 - v1)
    w1 = 1.0 / (1.0 + t)
    w2 = t * w1
    sig = jax.nn.sigmoid(eo)
    s1 = jnp.sum(jnp.where(m1, sig, 0.0), axis=1, keepdims=True)
    s2 = jnp.sum(jnp.where(m2, sig, 0.0), axis=1, keepdims=True)
    o_ref[...] = w1 * s1 + w2 * s2


def kernel(x, W_route, b_route, W_noise, b_noise, W_experts, b_experts):
    n, d = x.shape
    wcat = jnp.concatenate([W_noise, W_experts.T], axis=1)
    bcat = jnp.concatenate([b_noise, b_experts])[None, :]
    return pl.pallas_call(
        _moe_body,
        grid=(n // _BLK,),
        in_specs=[
            pl.BlockSpec((_BLK, d), lambda i: (i, 0)),
            pl.BlockSpec((d, 2 * _E), lambda i: (0, 0)),
            pl.BlockSpec((1, 2 * _E), lambda i: (0, 0)),
        ],
        out_specs=pl.BlockSpec((_BLK, 1), lambda i: (i, 0)),
        out_shape=jax.ShapeDtypeStruct((n, 1), jnp.float32),
    )(x, wcat, bcat)


# speedup vs baseline: 1.6963x; 1.6963x over previous
import jax
import jax.numpy as jnp
from jax.experimental import pallas as pl

_E = 16
_BLK = 1024


def _moe_body(x_ref, w_ref, b_ref, o_ref):
    xb = x_ref[...]
    wt = w_ref[...]
    acc = jax.lax.dot_general(wt, xb, (((1,), (1,)), ((), ())),
                              preferred_element_type=jnp.float32)
    acc = acc + b_ref[...]
    nz = acc[:_E, :]
    eo = acc[_E:, :]
    iota = jax.lax.broadcasted_iota(jnp.int32, nz.shape, 0).astype(jnp.float32)
    v1 = jnp.max(nz, axis=0, keepdims=True)
    i1 = jnp.min(jnp.where(nz == v1, iota, float(_E)), axis=0, keepdims=True)
    m1 = iota == i1
    masked = jnp.where(m1, -jnp.inf, nz)
    v2 = jnp.max(masked, axis=0, keepdims=True)
    i2 = jnp.min(jnp.where(masked == v2, iota, float(_E)), axis=0, keepdims=True)
    m2 = iota == i2
    t = jnp.exp(v2 - v1)
    rcp = 1.0 / (1.0 + t)
    w1 = rcp
    w2 = t * rcp
    sig = jax.nn.sigmoid(eo)
    coef = jnp.where(m1, w1, jnp.where(m2, w2, 0.0))
    o_ref[...] = jnp.sum(coef * sig, axis=0, keepdims=True)


def kernel(x, W_route, b_route, W_noise, b_noise, W_experts, b_experts):
    n, d = x.shape
    wt = jnp.concatenate([W_noise.T, W_experts], axis=0)
    bt = jnp.concatenate([b_noise, b_experts])[:, None]
    out = pl.pallas_call(
        _moe_body,
        grid=(n // _BLK,),
        in_specs=[
            pl.BlockSpec((_BLK, d), lambda i: (i, 0)),
            pl.BlockSpec((2 * _E, d), lambda i: (0, 0)),
            pl.BlockSpec((2 * _E, 1), lambda i: (0, 0)),
        ],
        out_specs=pl.BlockSpec((1, _BLK), lambda i: (0, i)),
        out_shape=jax.ShapeDtypeStruct((1, n), jnp.float32),
    )(x, wt, bt)
    return out.reshape(n, 1)
